# baseline (speedup 1.0000x reference)
import functools

import jax
import jax.numpy as jnp
from jax import lax
from jax.experimental import pallas as pl
from jax.experimental.pallas import tpu as pltpu
from jax.experimental.pallas import tpu_sc as plsc

NC = 2
NS = 16
LANES = 16
CHUNK = 128
DEG_W = 16
BR = 2000
NBUF = 4
LOOK = 2
UNROLL = 8


def _mesh():
    return plsc.VectorSubcoreMesh(core_axis_name="c", subcore_axis_name="s",
                                  num_cores=NC, num_subcores=NS)


_SC_PARAMS = pltpu.CompilerParams(use_tc_tiling_on_sc=False,
                                  needs_layout_passes=False)


@functools.lru_cache(maxsize=None)
def _deg_kernel(N, KCH):
    NT = N // NS

    @functools.partial(
        pl.kernel,
        out_type=jax.ShapeDtypeStruct((NC, N, DEG_W), jnp.float32),
        mesh=_mesh(),
        scratch_types=[
            pltpu.VMEM((KCH, CHUNK), jnp.int32),
            pltpu.VMEM((KCH, CHUNK), jnp.float32),
            pltpu.VMEM((CHUNK, DEG_W), jnp.float32),
            pltpu.VMEM_SHARED((N, DEG_W), jnp.float32),
        ],
        compiler_params=_SC_PARAMS,
    )
    def deg_kernel(dst_hbm, ew_hbm, out_hbm, dst_v, ew_v, vals, acc):
        cid = lax.axis_index("c")
        sid = lax.axis_index("s")
        zero = jnp.zeros((LANES,), jnp.float32)

        def zv(r, carry):
            vals[r, :] = zero
            return carry
        lax.fori_loop(0, CHUNK, zv, 0)

        base = sid * NT
        for k in range(NT // CHUNK):
            pltpu.sync_copy(vals, acc.at[pl.ds(base + k * CHUNK, CHUNK)])
        rem = NT % CHUNK
        if rem:
            pltpu.sync_copy(vals.at[pl.ds(0, rem)],
                            acc.at[pl.ds(base + (NT // CHUNK) * CHUNK, rem)])

        pltpu.sync_copy(dst_hbm.at[sid], dst_v)
        pltpu.sync_copy(ew_hbm.at[sid], ew_v)
        plsc.subcore_barrier()

        col0 = jnp.zeros((LANES,), jnp.int32)
        lane = lax.iota(jnp.int32, LANES)

        def chunk_body(jj, carry):
            j = 2 * jj + cid
            for r0 in range(0, CHUNK, LANES):
                w = ew_v[j, pl.ds(r0, LANES)]
                plsc.store_scatter(vals, [r0 + lane, col0], w)
            pltpu.sync_copy(vals, acc.at[dst_v.at[j]], add=True)
            return carry
        lax.fori_loop(0, KCH // 2, chunk_body, 0)

        plsc.subcore_barrier()
        pltpu.sync_copy(acc.at[pl.ds(base, NT)],
                        out_hbm.at[cid, pl.ds(base, NT)])

    return deg_kernel


NW = NC * NS


@functools.lru_cache(maxsize=None)
def _msg_kernel(N, D, KCH, q):
    NT = N // NS
    NQ = 2 * NC
    HD = D // NQ
    assert KCH % NBUF == 0

    @functools.partial(
        pl.kernel,
        out_type=jax.ShapeDtypeStruct((NC, N, HD), jnp.float32),
        mesh=_mesh(),
        scratch_types=[
            pltpu.VMEM((KCH, CHUNK), jnp.int32),
            pltpu.VMEM((KCH, CHUNK), jnp.int32),
            pltpu.VMEM((KCH, CHUNK), jnp.float32),
            pltpu.VMEM((NBUF, CHUNK, HD), jnp.float32),
            pltpu.VMEM_SHARED((N, HD), jnp.float32),
            pltpu.SemaphoreType.DMA,
            pltpu.SemaphoreType.DMA,
        ],
        compiler_params=_SC_PARAMS,
    )
    def msg_kernel(yv_hbm, src_hbm, dst_hbm, ew_hbm, out_hbm,
                   src_v, dst_v, ew_v, rows, acc, gsem, ssem):
        cid = lax.axis_index("c")
        sid = lax.axis_index("s")
        zero = jnp.zeros((LANES,), jnp.float32)

        def zr(r, carry):
            for k in range(HD // LANES):
                rows[0, r, pl.ds(k * LANES, LANES)] = zero
            return carry
        lax.fori_loop(0, CHUNK, zr, 0)

        base = sid * NT
        for k in range(NT // CHUNK):
            pltpu.sync_copy(rows.at[0], acc.at[pl.ds(base + k * CHUNK, CHUNK)])
        rem = NT % CHUNK
        if rem:
            pltpu.sync_copy(rows.at[0, pl.ds(0, rem)],
                            acc.at[pl.ds(base + (NT // CHUNK) * CHUNK, rem)])

        pltpu.sync_copy(src_hbm.at[sid], src_v)
        pltpu.sync_copy(dst_hbm.at[sid], dst_v)
        pltpu.sync_copy(ew_hbm.at[sid], ew_v)

        def xf(j, carry):
            for k in range(CHUNK // LANES):
                sl = pl.ds(k * LANES, LANES)
                src_v[j, sl] = src_v[j, sl] * NQ + (NC * q + cid)
            return carry
        lax.fori_loop(0, KCH, xf, 0)
        plsc.subcore_barrier()

        def gissue(j, b):
            return pltpu.async_copy(yv_hbm.at[src_v.at[j]], rows.at[b], gsem)

        def scat(j, b):
            pltpu.sync_copy(rows.at[b], acc.at[dst_v.at[j]], add=True)

        def scale(j, b):
            jf = jnp.full((LANES,), j, jnp.int32)

            def srow(rr, c2):
                for u in range(UNROLL):
                    r = rr * UNROLL + u
                    w = plsc.load_gather(
                        ew_v, [jf, jnp.full((LANES,), r, jnp.int32)])
                    for k in range(HD // LANES):
                        sl = pl.ds(k * LANES, LANES)
                        rows[b, r, sl] = rows[b, r, sl] * w
                return c2
            lax.fori_loop(0, CHUNK // UNROLL, srow, 0)

        def group(jj, carry):
            j0 = jj * NBUF
            gissue(j0, 0).wait()
            for b in range(NBUF):
                j = j0 + b
                dnext = gissue(j + 1, b + 1) if b + 1 < NBUF else None
                scale(j, b)
                scat(j, b)
                if dnext is not None:
                    dnext.wait()
            return carry
        lax.fori_loop(0, KCH // NBUF, group, 0)
        plsc.subcore_barrier()
        pltpu.sync_copy(acc.at[pl.ds(base, NT)],
                        out_hbm.at[cid, pl.ds(base, NT)])

    return msg_kernel


def _dinv_block(p_ref):
    deg = p_ref[0, :, 0:1] + p_ref[1, :, 0:1] + 1.0
    return lax.rsqrt(deg)


def _cat(m0_ref, m1_ref):
    return jnp.concatenate([m0_ref[0], m0_ref[1], m1_ref[0], m1_ref[1]],
                           axis=-1)


def _tc_y(p, X, W):
    N, D = X.shape

    def body(p_ref, x_ref, w_ref, y_ref):
        y_ref[...] = _dinv_block(p_ref) * jnp.dot(
            x_ref[...], w_ref[...], preferred_element_type=jnp.float32)

    return pl.pallas_call(
        body,
        grid=(N // BR,),
        in_specs=[
            pl.BlockSpec((2, BR, DEG_W), lambda i: (0, i, 0)),
            pl.BlockSpec((BR, D), lambda i: (i, 0)),
            pl.BlockSpec((D, D), lambda i: (0, 0)),
        ],
        out_specs=pl.BlockSpec((BR, D), lambda i: (i, 0)),
        out_shape=jax.ShapeDtypeStruct((N, D), jnp.float32),
    )(p, X, W)


def _tc_mid(p, m0, m1, Y1, b1, W2):
    N, D = Y1.shape

    def body(p_ref, m0_ref, m1_ref, y1_ref, b_ref, w_ref, y2_ref):
        dinv = _dinv_block(p_ref)
        h = jnp.maximum(dinv * (_cat(m0_ref, m1_ref) + y1_ref[...])
                        + b_ref[...], 0.0)
        y2_ref[...] = dinv * jnp.dot(h, w_ref[...],
                                     preferred_element_type=jnp.float32)

    return pl.pallas_call(
        body,
        grid=(N // BR,),
        in_specs=[
            pl.BlockSpec((2, BR, DEG_W), lambda i: (0, i, 0)),
            pl.BlockSpec((2, BR, D // 4), lambda i: (0, i, 0)),
            pl.BlockSpec((2, BR, D // 4), lambda i: (0, i, 0)),
            pl.BlockSpec((BR, D), lambda i: (i, 0)),
            pl.BlockSpec((1, D), lambda i: (0, 0)),
            pl.BlockSpec((D, D), lambda i: (0, 0)),
        ],
        out_specs=pl.BlockSpec((BR, D), lambda i: (i, 0)),
        out_shape=jax.ShapeDtypeStruct((N, D), jnp.float32),
    )(p, m0, m1, Y1, b1, W2)


def _tc_out(p, r0, r1, Y2, b2):
    N, D = Y2.shape

    def body(p_ref, r0_ref, r1_ref, y2_ref, b_ref, o_ref):
        dinv = _dinv_block(p_ref)
        o_ref[...] = jnp.maximum(
            dinv * (_cat(r0_ref, r1_ref) + y2_ref[...]) + b_ref[...], 0.0)

    return pl.pallas_call(
        body,
        grid=(N // BR,),
        in_specs=[
            pl.BlockSpec((2, BR, DEG_W), lambda i: (0, i, 0)),
            pl.BlockSpec((2, BR, D // 4), lambda i: (0, i, 0)),
            pl.BlockSpec((2, BR, D // 4), lambda i: (0, i, 0)),
            pl.BlockSpec((BR, D), lambda i: (i, 0)),
            pl.BlockSpec((1, D), lambda i: (0, 0)),
        ],
        out_specs=pl.BlockSpec((BR, D), lambda i: (i, 0)),
        out_shape=jax.ShapeDtypeStruct((N, D), jnp.float32),
    )(p, r0, r1, Y2, b2)


def kernel(X, edge_index, edge_weight, W1, b1, W2, b2):
    N, D = X.shape
    E = edge_weight.shape[0]
    KCH = -(-E // (NS * CHUNK))
    KCH = -(-KCH // (2 * NBUF)) * (2 * NBUF)
    EP = NS * KCH * CHUNK
    pad = EP - E

    src = jnp.pad(edge_index[0], (0, pad)).reshape(NS, KCH, CHUNK)
    dst = jnp.pad(edge_index[1], (0, pad)).reshape(NS, KCH, CHUNK)
    ew = jnp.pad(edge_weight, (0, pad)).reshape(NS, KCH, CHUNK)

    NQ = 2 * NC
    p = _deg_kernel(N, KCH)(dst, ew)
    Y1 = _tc_y(p, X, W1)
    y1v = Y1.reshape(N * NQ, D // NQ)
    m0 = _msg_kernel(N, D, KCH, 0)(y1v, src, dst, ew)
    m1 = _msg_kernel(N, D, KCH, 1)(y1v, src, dst, ew)
    Y2 = _tc_mid(p, m0, m1, Y1, b1.reshape(1, D), W2)
    y2v = Y2.reshape(N * NQ, D // NQ)
    r0 = _msg_kernel(N, D, KCH, 0)(y2v, src, dst, ew)
    r1 = _msg_kernel(N, D, KCH, 1)(y2v, src, dst, ew)
    return _tc_out(p, r0, r1, Y2, b2.reshape(1, D))


# speedup vs baseline: 1.4071x; 1.4071x over previous
import functools

import jax
import jax.numpy as jnp
from jax import lax
from jax.experimental import pallas as pl
from jax.experimental.pallas import tpu as pltpu
from jax.experimental.pallas import tpu_sc as plsc

NC = 2
NS = 16
NW = NC * NS
LANES = 16
CHUNK = 128
DEG_W = 16
BR = 2000
UNROLL = 8


def _mesh():
    return plsc.VectorSubcoreMesh(core_axis_name="c", subcore_axis_name="s",
                                  num_cores=NC, num_subcores=NS)


_SC_PARAMS = pltpu.CompilerParams(use_tc_tiling_on_sc=False,
                                  needs_layout_passes=False)


@functools.lru_cache(maxsize=None)
def _deg_kernel(N, KCH):
    NT = N // NS

    @functools.partial(
        pl.kernel,
        out_type=jax.ShapeDtypeStruct((NC, N, DEG_W), jnp.float32),
        mesh=_mesh(),
        scratch_types=[
            pltpu.VMEM((KCH, CHUNK), jnp.int32),
            pltpu.VMEM((KCH, CHUNK), jnp.float32),
            pltpu.VMEM((CHUNK, DEG_W), jnp.float32),
            pltpu.VMEM_SHARED((N, DEG_W), jnp.float32),
        ],
        compiler_params=_SC_PARAMS,
    )
    def deg_kernel(dst_hbm, ew_hbm, out_hbm, dst_v, ew_v, vals, acc):
        cid = lax.axis_index("c")
        sid = lax.axis_index("s")
        wid = sid * NC + cid
        zero = jnp.zeros((LANES,), jnp.float32)

        def zv(r, carry):
            vals[r, :] = zero
            return carry
        lax.fori_loop(0, CHUNK, zv, 0)

        base = sid * NT
        for k in range(NT // CHUNK):
            pltpu.sync_copy(vals, acc.at[pl.ds(base + k * CHUNK, CHUNK)])
        rem = NT % CHUNK
        if rem:
            pltpu.sync_copy(vals.at[pl.ds(0, rem)],
                            acc.at[pl.ds(base + (NT // CHUNK) * CHUNK, rem)])

        pltpu.sync_copy(dst_hbm.at[wid], dst_v)
        pltpu.sync_copy(ew_hbm.at[wid], ew_v)
        plsc.subcore_barrier()

        col0 = jnp.zeros((LANES,), jnp.int32)
        lane = lax.iota(jnp.int32, LANES)

        def chunk_body(j, carry):
            for r0 in range(0, CHUNK, LANES):
                w = ew_v[j, pl.ds(r0, LANES)]
                plsc.store_scatter(vals, [r0 + lane, col0], w)
            pltpu.sync_copy(vals, acc.at[dst_v.at[j]], add=True)
            return carry
        lax.fori_loop(0, KCH, chunk_body, 0)

        plsc.subcore_barrier()
        pltpu.sync_copy(acc.at[pl.ds(base, NT)],
                        out_hbm.at[cid, pl.ds(base, NT)])

    return deg_kernel


@functools.lru_cache(maxsize=None)
def _msg_kernel(N, D, KCH):
    NT = N // NS

    @functools.partial(
        pl.kernel,
        out_type=jax.ShapeDtypeStruct((NC, N, D), jnp.float32),
        mesh=_mesh(),
        scratch_types=[
            pltpu.VMEM((KCH, CHUNK), jnp.int32),
            pltpu.VMEM((KCH, CHUNK), jnp.int32),
            pltpu.VMEM((KCH, CHUNK), jnp.float32),
            pltpu.VMEM((CHUNK, D), jnp.float32),
            pltpu.VMEM_SHARED((N, D), jnp.float32),
            pltpu.SemaphoreType.DMA,
        ],
        compiler_params=_SC_PARAMS,
    )
    def msg_kernel(y_hbm, src_hbm, dst_hbm, ew_hbm, out_hbm,
                   src_v, dst_v, ew_v, rows, acc, sem):
        cid = lax.axis_index("c")
        sid = lax.axis_index("s")
        wid = sid * NC + cid
        zero = jnp.zeros((LANES,), jnp.float32)

        def zr(r, carry):
            for k in range(D // LANES):
                rows[r, pl.ds(k * LANES, LANES)] = zero
            return carry
        lax.fori_loop(0, CHUNK, zr, 0)

        base = sid * NT
        for k in range(NT // CHUNK):
            pltpu.sync_copy(rows, acc.at[pl.ds(base + k * CHUNK, CHUNK)])
        rem = NT % CHUNK
        if rem:
            pltpu.sync_copy(rows.at[pl.ds(0, rem)],
                            acc.at[pl.ds(base + (NT // CHUNK) * CHUNK, rem)])

        pltpu.sync_copy(src_hbm.at[wid], src_v)
        pltpu.sync_copy(dst_hbm.at[wid], dst_v)
        pltpu.sync_copy(ew_hbm.at[wid], ew_v)
        plsc.subcore_barrier()

        def chunk_body(j, carry):
            pltpu.async_copy(y_hbm.at[src_v.at[j]], rows, sem).wait()
            jf = jnp.full((LANES,), j, jnp.int32)

            def srow(rr, c2):
                for u in range(UNROLL):
                    r = rr * UNROLL + u
                    w = plsc.load_gather(
                        ew_v, [jf, jnp.full((LANES,), r, jnp.int32)])
                    for k in range(D // LANES):
                        sl = pl.ds(k * LANES, LANES)
                        rows[r, sl] = rows[r, sl] * w
                return c2
            lax.fori_loop(0, CHUNK // UNROLL, srow, 0)
            pltpu.sync_copy(rows, acc.at[dst_v.at[j]], add=True)
            return carry
        lax.fori_loop(0, KCH, chunk_body, 0)

        plsc.subcore_barrier()
        pltpu.sync_copy(acc.at[pl.ds(base, NT)],
                        out_hbm.at[cid, pl.ds(base, NT)])

    return msg_kernel


def _dinv_block(p_ref):
    deg = p_ref[0, :, 0:1] + p_ref[1, :, 0:1] + 1.0
    return lax.rsqrt(deg)


def _psum(m_ref):
    return m_ref[0] + m_ref[1]


def _tc_y(p, X, W):
    N, D = X.shape

    def body(p_ref, x_ref, w_ref, y_ref):
        y_ref[...] = _dinv_block(p_ref) * jnp.dot(
            x_ref[...], w_ref[...], preferred_element_type=jnp.float32)

    return pl.pallas_call(
        body,
        grid=(N // BR,),
        in_specs=[
            pl.BlockSpec((2, BR, DEG_W), lambda i: (0, i, 0)),
            pl.BlockSpec((BR, D), lambda i: (i, 0)),
            pl.BlockSpec((D, D), lambda i: (0, 0)),
        ],
        out_specs=pl.BlockSpec((BR, D), lambda i: (i, 0)),
        out_shape=jax.ShapeDtypeStruct((N, D), jnp.float32),
    )(p, X, W)


def _tc_mid(p, m, Y1, b1, W2):
    N, D = Y1.shape

    def body(p_ref, m_ref, y1_ref, b_ref, w_ref, y2_ref):
        dinv = _dinv_block(p_ref)
        h = jnp.maximum(dinv * (_psum(m_ref) + y1_ref[...]) + b_ref[...], 0.0)
        y2_ref[...] = dinv * jnp.dot(h, w_ref[...],
                                     preferred_element_type=jnp.float32)

    return pl.pallas_call(
        body,
        grid=(N // BR,),
        in_specs=[
            pl.BlockSpec((2, BR, DEG_W), lambda i: (0, i, 0)),
            pl.BlockSpec((2, BR, D), lambda i: (0, i, 0)),
            pl.BlockSpec((BR, D), lambda i: (i, 0)),
            pl.BlockSpec((1, D), lambda i: (0, 0)),
            pl.BlockSpec((D, D), lambda i: (0, 0)),
        ],
        out_specs=pl.BlockSpec((BR, D), lambda i: (i, 0)),
        out_shape=jax.ShapeDtypeStruct((N, D), jnp.float32),
    )(p, m, Y1, b1, W2)


def _tc_out(p, r, Y2, b2):
    N, D = Y2.shape

    def body(p_ref, r_ref, y2_ref, b_ref, o_ref):
        dinv = _dinv_block(p_ref)
        o_ref[...] = jnp.maximum(
            dinv * (_psum(r_ref) + y2_ref[...]) + b_ref[...], 0.0)

    return pl.pallas_call(
        body,
        grid=(N // BR,),
        in_specs=[
            pl.BlockSpec((2, BR, DEG_W), lambda i: (0, i, 0)),
            pl.BlockSpec((2, BR, D), lambda i: (0, i, 0)),
            pl.BlockSpec((BR, D), lambda i: (i, 0)),
            pl.BlockSpec((1, D), lambda i: (0, 0)),
        ],
        out_specs=pl.BlockSpec((BR, D), lambda i: (i, 0)),
        out_shape=jax.ShapeDtypeStruct((N, D), jnp.float32),
    )(p, r, Y2, b2)


def kernel(X, edge_index, edge_weight, W1, b1, W2, b2):
    N, D = X.shape
    E = edge_weight.shape[0]
    KCH = -(-E // (NW * CHUNK))
    EP = NW * KCH * CHUNK
    pad = EP - E

    src = jnp.pad(edge_index[0], (0, pad)).reshape(NW, KCH, CHUNK)
    dst = jnp.pad(edge_index[1], (0, pad)).reshape(NW, KCH, CHUNK)
    ew = jnp.pad(edge_weight, (0, pad)).reshape(NW, KCH, CHUNK)

    p = _deg_kernel(N, KCH)(dst, ew)
    Y1 = _tc_y(p, X, W1)
    m = _msg_kernel(N, D, KCH)(Y1, src, dst, ew)
    Y2 = _tc_mid(p, m, Y1, b1.reshape(1, D), W2)
    r = _msg_kernel(N, D, KCH)(Y2, src, dst, ew)
    return _tc_out(p, r, Y2, b2.reshape(1, D))
